# Initial kernel scaffold; baseline (speedup 1.0000x reference)
import jax
import jax.numpy as jnp
from jax.experimental import pallas as pl


def kernel(lt_inputs, lt_offsets, gt_inputs, gt_offsets, show_ids, lt_tables, gt_tables, show_table, lin_w, lin_b):
    raise NotImplementedError("write your pallas kernel here")


# speedup vs baseline: 12.8159x; 12.8159x over previous
import functools

import jax
import jax.numpy as jnp
from jax import lax
from jax.experimental import pallas as pl
from jax.experimental.pallas import tpu as pltpu
from jax.experimental.pallas import tpu_sc as plsc

_HASH = 21000
_NLT = 6
_NGT = 5
_NPOOL = _NLT + _NGT
_NE = _NPOOL + 1
_NLBL = 6
_B = 16384
_L = 20
_D = 64
_NPAIR = _NE * (_NE + 1) // 2
_K2 = _NPAIR * _D + _NE * _D

_NC, _NS = 2, 16
_NW = _NC * _NS
_BAGS_PER_W = _NPOOL * _B // _NW
_CHUNK_BAGS = 32
_CHUNKS = _BAGS_PER_W // _CHUNK_BAGS
_IDX_ROWS = _CHUNK_BAGS * _L // 128
_SHOW_CHUNKS = _B // _NW // 128

_PAIRS = [(i, j) for i in range(_NE) for j in range(i, _NE)]


def _sc_body(tab_hbm, idx_hbm, sidx_hbm, out_hbm, idx_v, sidx_v, rows_v, out_v, sem):
    wid = lax.axis_index("s") * _NC + lax.axis_index("c")

    @pl.loop(0, _CHUNKS)
    def _pooled(c):
        g0 = wid * _BAGS_PER_W + c * _CHUNK_BAGS
        t = g0 >> 14
        b0 = pl.multiple_of(g0 & (_B - 1), _CHUNK_BAGS)
        i0 = pl.multiple_of(g0 * _L, _CHUNK_BAGS * _L)
        pltpu.sync_copy(idx_hbm.at[pl.ds(i0, _CHUNK_BAGS * _L)], idx_v)
        cps = [
            pltpu.async_copy(tab_hbm.at[idx_v.at[pl.ds(j * 128, 128)]],
                             rows_v.at[pl.ds(j * 128, 128)], sem)
            for j in range(_IDX_ROWS)
        ]
        for cp in cps:
            cp.wait()

        @pl.loop(0, _CHUNK_BAGS)
        def _bag(bi):
            rr = bi * _L
            accs = [rows_v[rr, pl.ds(cc * 16, 16)] for cc in range(4)]
            for l in range(1, _L):
                accs = [jnp.maximum(accs[cc], rows_v[rr + l, pl.ds(cc * 16, 16)])
                        for cc in range(4)]
            for cc in range(4):
                out_v[bi, pl.ds(cc * 16, 16)] = accs[cc]

        pltpu.sync_copy(out_v, out_hbm.at[t, pl.ds(b0, _CHUNK_BAGS)])

    @pl.loop(0, _SHOW_CHUNKS)
    def _show(j):
        rrow = wid * _SHOW_CHUNKS + j
        s0 = pl.multiple_of(rrow * 128, 128)
        pltpu.sync_copy(sidx_hbm.at[pl.ds(s0, 128)], sidx_v)
        pltpu.async_copy(tab_hbm.at[sidx_v],
                         rows_v.at[pl.ds(0, 128)], sem).wait()
        pltpu.sync_copy(rows_v.at[pl.ds(0, 128)],
                        out_hbm.at[_NPOOL, pl.ds(s0, 128)])


def _tc_body(e_ref, wh_ref, wl_ref, b_ref, o_ref, zh_ref, zl_ref):
    es = [e_ref[i] for i in range(_NE)]

    def put(col, val):
        hi = val.astype(jnp.bfloat16)
        lo = (val - hi.astype(jnp.float32)).astype(jnp.bfloat16)
        zh_ref[:, pl.ds(col, _D)] = hi
        zl_ref[:, pl.ds(col, _D)] = lo

    for p, (i, j) in enumerate(_PAIRS):
        put(p * _D, es[i] * es[j])
    for t in range(_NE):
        put(_NPAIR * _D + t * _D, es[t])
    zh, zl = zh_ref[...], zl_ref[...]
    o_ref[...] = (
        jnp.dot(zh, wh_ref[...], preferred_element_type=jnp.float32)
        + jnp.dot(zh, wl_ref[...], preferred_element_type=jnp.float32)
        + jnp.dot(zl, wh_ref[...], preferred_element_type=jnp.float32)
        + b_ref[...]
    )


def _sc_embed(tab, idx2, sidx):
    mesh = plsc.VectorSubcoreMesh(core_axis_name="c", subcore_axis_name="s",
                                  num_cores=_NC, num_subcores=_NS)
    run = pl.kernel(
        _sc_body,
        out_type=jax.ShapeDtypeStruct((_NE, _B, _D), jnp.float32),
        mesh=mesh,
        compiler_params=pltpu.CompilerParams(use_tc_tiling_on_sc=False),
        scratch_types=[
            pltpu.VMEM((_CHUNK_BAGS * _L,), jnp.int32),
            pltpu.VMEM((128,), jnp.int32),
            pltpu.VMEM((_CHUNK_BAGS * _L, _D), jnp.float32),
            pltpu.VMEM((_CHUNK_BAGS, _D), jnp.float32),
            pltpu.SemaphoreType.DMA,
        ],
    )
    return run(tab, idx2, sidx)


def _tc_interact(embeds, w2, b2, bt=512):
    wh = w2.astype(jnp.bfloat16)
    wl = (w2 - wh.astype(jnp.float32)).astype(jnp.bfloat16)
    return pl.pallas_call(
        _tc_body,
        grid=(_B // bt,),
        in_specs=[
            pl.BlockSpec((_NE, bt, _D), lambda i: (0, i, 0)),
            pl.BlockSpec((_K2, _NLBL), lambda i: (0, 0)),
            pl.BlockSpec((_K2, _NLBL), lambda i: (0, 0)),
            pl.BlockSpec((1, _NLBL), lambda i: (0, 0)),
        ],
        out_specs=pl.BlockSpec((bt, _NLBL), lambda i: (i, 0)),
        out_shape=jax.ShapeDtypeStruct((_B, _NLBL), jnp.float32),
        scratch_shapes=[pltpu.VMEM((bt, _K2), jnp.bfloat16),
                        pltpu.VMEM((bt, _K2), jnp.bfloat16)],
    )(embeds, wh, wl, b2)


def kernel(lt_inputs, lt_offsets, gt_inputs, gt_offsets, show_ids,
           lt_tables, gt_tables, show_table, lin_w, lin_b):
    tab = jnp.concatenate(
        [lt_tables.reshape(-1, _D), gt_tables.reshape(-1, _D), show_table], axis=0)
    idx = (jnp.concatenate([lt_inputs, gt_inputs], axis=0)
           + (jnp.arange(_NPOOL, dtype=jnp.int32) * _HASH)[:, None])
    idx2 = idx.reshape(-1)
    sidx = show_ids + _NPOOL * _HASH
    embeds = _sc_embed(tab, idx2, sidx)

    w2 = jnp.concatenate(
        [jnp.repeat(lin_w[:, :_NPAIR], _D, axis=1) * (1.0 / _D),
         lin_w[:, _NPAIR:]], axis=1).T
    out2 = _tc_interact(embeds, w2, lin_b.reshape(1, _NLBL))
    return tuple(out2[:, k] for k in range(_NLBL))


# speedup vs baseline: 18.3837x; 1.4344x over previous
import functools

import jax
import jax.numpy as jnp
from jax import lax
from jax.experimental import pallas as pl
from jax.experimental.pallas import tpu as pltpu
from jax.experimental.pallas import tpu_sc as plsc

_HASH = 21000
_NLT = 6
_NGT = 5
_NPOOL = _NLT + _NGT
_NE = _NPOOL + 1
_NLBL = 6
_B = 16384
_L = 20
_D = 64
_NPAIR = _NE * (_NE + 1) // 2
_K2 = _NPAIR * _D + _NE * _D

_NC, _NS = 2, 16
_NW = _NC * _NS
_BAGS_PER_W = _NPOOL * _B // _NW
_CHUNK_BAGS = 32
_CHUNKS = _BAGS_PER_W // _CHUNK_BAGS
_IDX_ROWS = _CHUNK_BAGS * _L // 128
_SHOW_CHUNKS = _B // _NW // 128

_PAIRS = [(i, j) for i in range(_NE) for j in range(i, _NE)]


def _sc_body(tab_hbm, idx_hbm, sidx_hbm, out_hbm,
             idx_a, idx_b, sidx_v, rows_a, rows_b, out_a, out_b,
             si_a, si_b, sg_a, sg_b, so_a, so_b, sem):
    wid = lax.axis_index("s") * _NC + lax.axis_index("c")
    nidx = _CHUNK_BAGS * _L

    def idx_base(c):
        g0 = wid * _BAGS_PER_W + c * _CHUNK_BAGS
        return pl.multiple_of(g0 * _L, nidx)

    def start_idx(c, idx_v, si):
        pltpu.async_copy(idx_hbm.at[pl.ds(idx_base(c), nidx)], idx_v, si)

    def wait_idx(idx_v, si):
        pltpu.make_async_copy(idx_hbm.at[pl.ds(0, nidx)], idx_v, si).wait()

    def fire_gathers(idx_v, rows_v, sg):
        for j in range(_IDX_ROWS):
            pltpu.async_copy(tab_hbm.at[idx_v.at[pl.ds(j * 128, 128)]],
                             rows_v.at[pl.ds(j * 128, 128)], sg)

    def wait_gathers(rows_v, sg):
        pltpu.make_async_copy(tab_hbm.at[pl.ds(0, nidx)], rows_v, sg).wait()

    def process(c, rows_v, out_v, so):
        @pl.loop(0, _CHUNK_BAGS)
        def _bag(bi):
            rr = bi * _L
            accs = [rows_v[rr, pl.ds(cc * 16, 16)] for cc in range(4)]
            for l in range(1, _L):
                accs = [jnp.maximum(accs[cc], rows_v[rr + l, pl.ds(cc * 16, 16)])
                        for cc in range(4)]
            for cc in range(4):
                out_v[bi, pl.ds(cc * 16, 16)] = accs[cc]

        g0 = wid * _BAGS_PER_W + c * _CHUNK_BAGS
        t = g0 >> 14
        b0 = pl.multiple_of(g0 & (_B - 1), _CHUNK_BAGS)
        pltpu.async_copy(out_v, out_hbm.at[t, pl.ds(b0, _CHUNK_BAGS)], so)

    def wait_out(out_v, so):
        pltpu.make_async_copy(out_v, out_hbm.at[0, pl.ds(0, _CHUNK_BAGS)], so).wait()

    start_idx(0, idx_a, si_a)
    wait_idx(idx_a, si_a)
    fire_gathers(idx_a, rows_a, sg_a)
    start_idx(1, idx_b, si_b)

    @pl.loop(0, _CHUNKS, step=2)
    def _pooled(c):
        wait_idx(idx_b, si_b)
        fire_gathers(idx_b, rows_b, sg_b)

        wait_gathers(rows_a, sg_a)

        @pl.when(c + 2 < _CHUNKS)
        def _():
            start_idx(c + 2, idx_a, si_a)

        @pl.when(c >= 2)
        def _():
            wait_out(out_a, so_a)
        process(c, rows_a, out_a, so_a)

        @pl.when(c + 2 < _CHUNKS)
        def _():
            wait_idx(idx_a, si_a)
            fire_gathers(idx_a, rows_a, sg_a)

        wait_gathers(rows_b, sg_b)

        @pl.when(c + 3 < _CHUNKS)
        def _():
            start_idx(c + 3, idx_b, si_b)

        @pl.when(c >= 2)
        def _():
            wait_out(out_b, so_b)
        process(c + 1, rows_b, out_b, so_b)

    wait_out(out_a, so_a)
    wait_out(out_b, so_b)

    @pl.loop(0, _SHOW_CHUNKS)
    def _show(j):
        rrow = wid * _SHOW_CHUNKS + j
        s0 = pl.multiple_of(rrow * 128, 128)
        pltpu.sync_copy(sidx_hbm.at[pl.ds(s0, 128)], sidx_v)
        pltpu.async_copy(tab_hbm.at[sidx_v],
                         rows_a.at[pl.ds(0, 128)], sem).wait()
        pltpu.sync_copy(rows_a.at[pl.ds(0, 128)],
                        out_hbm.at[_NPOOL, pl.ds(s0, 128)])


def _tc_body(e_ref, wh_ref, wl_ref, b_ref, o_ref, zh_ref, zl_ref):
    es = [e_ref[i] for i in range(_NE)]

    def put(col, val):
        hi = val.astype(jnp.bfloat16)
        lo = (val - hi.astype(jnp.float32)).astype(jnp.bfloat16)
        zh_ref[:, pl.ds(col, _D)] = hi
        zl_ref[:, pl.ds(col, _D)] = lo

    for p, (i, j) in enumerate(_PAIRS):
        put(p * _D, es[i] * es[j])
    for t in range(_NE):
        put(_NPAIR * _D + t * _D, es[t])
    zh, zl = zh_ref[...], zl_ref[...]
    o_ref[...] = (
        jnp.dot(zh, wh_ref[...], preferred_element_type=jnp.float32)
        + jnp.dot(zh, wl_ref[...], preferred_element_type=jnp.float32)
        + jnp.dot(zl, wh_ref[...], preferred_element_type=jnp.float32)
        + b_ref[...]
    )


def _sc_embed(tab, idx2, sidx):
    mesh = plsc.VectorSubcoreMesh(core_axis_name="c", subcore_axis_name="s",
                                  num_cores=_NC, num_subcores=_NS)
    run = pl.kernel(
        _sc_body,
        out_type=jax.ShapeDtypeStruct((_NE, _B, _D), jnp.float32),
        mesh=mesh,
        compiler_params=pltpu.CompilerParams(use_tc_tiling_on_sc=False),
        scratch_types=[
            pltpu.VMEM((_CHUNK_BAGS * _L,), jnp.int32),
            pltpu.VMEM((_CHUNK_BAGS * _L,), jnp.int32),
            pltpu.VMEM((128,), jnp.int32),
            pltpu.VMEM((_CHUNK_BAGS * _L, _D), jnp.float32),
            pltpu.VMEM((_CHUNK_BAGS * _L, _D), jnp.float32),
            pltpu.VMEM((_CHUNK_BAGS, _D), jnp.float32),
            pltpu.VMEM((_CHUNK_BAGS, _D), jnp.float32),
            pltpu.SemaphoreType.DMA,
            pltpu.SemaphoreType.DMA,
            pltpu.SemaphoreType.DMA,
            pltpu.SemaphoreType.DMA,
            pltpu.SemaphoreType.DMA,
            pltpu.SemaphoreType.DMA,
            pltpu.SemaphoreType.DMA,
        ],
    )
    return run(tab, idx2, sidx)


def _tc_interact(embeds, w2, b2, bt=512):
    wh = w2.astype(jnp.bfloat16)
    wl = (w2 - wh.astype(jnp.float32)).astype(jnp.bfloat16)
    return pl.pallas_call(
        _tc_body,
        grid=(_B // bt,),
        in_specs=[
            pl.BlockSpec((_NE, bt, _D), lambda i: (0, i, 0)),
            pl.BlockSpec((_K2, _NLBL), lambda i: (0, 0)),
            pl.BlockSpec((_K2, _NLBL), lambda i: (0, 0)),
            pl.BlockSpec((1, _NLBL), lambda i: (0, 0)),
        ],
        out_specs=pl.BlockSpec((bt, _NLBL), lambda i: (i, 0)),
        out_shape=jax.ShapeDtypeStruct((_B, _NLBL), jnp.float32),
        scratch_shapes=[pltpu.VMEM((bt, _K2), jnp.bfloat16),
                        pltpu.VMEM((bt, _K2), jnp.bfloat16)],
    )(embeds, wh, wl, b2)


def kernel(lt_inputs, lt_offsets, gt_inputs, gt_offsets, show_ids,
           lt_tables, gt_tables, show_table, lin_w, lin_b):
    tab = jnp.concatenate(
        [lt_tables.reshape(-1, _D), gt_tables.reshape(-1, _D), show_table], axis=0)
    idx = (jnp.concatenate([lt_inputs, gt_inputs], axis=0)
           + (jnp.arange(_NPOOL, dtype=jnp.int32) * _HASH)[:, None])
    idx2 = idx.reshape(-1)
    sidx = show_ids + _NPOOL * _HASH
    embeds = _sc_embed(tab, idx2, sidx)

    w2 = jnp.concatenate(
        [jnp.repeat(lin_w[:, :_NPAIR], _D, axis=1) * (1.0 / _D),
         lin_w[:, _NPAIR:]], axis=1).T
    out2 = _tc_interact(embeds, w2, lin_b.reshape(1, _NLBL))
    return tuple(out2[:, k] for k in range(_NLBL))


# speedup vs baseline: 20.6330x; 1.1224x over previous
import functools

import jax
import jax.numpy as jnp
from jax import lax
from jax.experimental import pallas as pl
from jax.experimental.pallas import tpu as pltpu
from jax.experimental.pallas import tpu_sc as plsc

_HASH = 21000
_NLT = 6
_NGT = 5
_NPOOL = _NLT + _NGT
_NE = _NPOOL + 1
_NLBL = 6
_B = 16384
_L = 20
_D = 64
_NPAIR = _NE * (_NE + 1) // 2
_K2 = _NPAIR * _D + _NE * _D

_NC, _NS = 2, 16
_NW = _NC * _NS
_BAGS_PER_W = _NPOOL * _B // _NW
_CHUNK_BAGS = 32
_CHUNKS = _BAGS_PER_W // _CHUNK_BAGS
_IDX_ROWS = _CHUNK_BAGS * _L // 128
_SHOW_CHUNKS = _B // _NW // 128

_PAIRS = [(i, j) for i in range(_NE) for j in range(i, _NE)]


def _sc_body(lt_hbm, gt_hbm, show_hbm, lidx_hbm, gidx_hbm, sidx_hbm, out_hbm,
             idx_a, idx_b, sidx_v, rows_a, rows_b, out_a, out_b,
             si_a, si_b, sg_a, sg_b, so_a, so_b, sem):
    wid = lax.axis_index("s") * _NC + lax.axis_index("c")
    nidx = _CHUNK_BAGS * _L

    def wait_idx(idx_v, si):
        pltpu.make_async_copy(lidx_hbm.at[pl.ds(0, nidx)], idx_v, si).wait()

    def wait_gathers(rows_v, sg):
        pltpu.make_async_copy(show_hbm.at[pl.ds(0, nidx)], rows_v, sg).wait()

    def wait_out(out_v, so):
        pltpu.make_async_copy(out_v, out_hbm.at[0, pl.ds(0, _CHUNK_BAGS)], so).wait()

    def section(tab_hbm, idx_hbm, plane_base, ntab):
        bags_per_w = ntab * _B // _NW
        chunks = bags_per_w // _CHUNK_BAGS

        def start_idx(c, idx_v, si):
            g0 = wid * bags_per_w + c * _CHUNK_BAGS
            i0 = pl.multiple_of(g0 * _L, nidx)
            pltpu.async_copy(idx_hbm.at[pl.ds(i0, nidx)], idx_v, si)

        def fire_gathers(c, idx_v, rows_v, sg):
            g0 = wid * bags_per_w + c * _CHUNK_BAGS
            t = g0 >> 14
            for j in range(_IDX_ROWS):
                pltpu.async_copy(tab_hbm.at[t].at[idx_v.at[pl.ds(j * 128, 128)]],
                                 rows_v.at[pl.ds(j * 128, 128)], sg)

        def process(c, rows_v, out_v, so):
            @pl.loop(0, _CHUNK_BAGS)
            def _bag(bi):
                rr = bi * _L
                accs = [rows_v[rr, pl.ds(cc * 16, 16)] for cc in range(4)]
                for l in range(1, _L):
                    accs = [jnp.maximum(accs[cc], rows_v[rr + l, pl.ds(cc * 16, 16)])
                            for cc in range(4)]
                for cc in range(4):
                    out_v[bi, pl.ds(cc * 16, 16)] = accs[cc]

            g0 = wid * bags_per_w + c * _CHUNK_BAGS
            t = g0 >> 14
            b0 = pl.multiple_of(g0 & (_B - 1), _CHUNK_BAGS)
            pltpu.async_copy(out_v, out_hbm.at[plane_base + t, pl.ds(b0, _CHUNK_BAGS)], so)

        start_idx(0, idx_a, si_a)
        wait_idx(idx_a, si_a)
        fire_gathers(0, idx_a, rows_a, sg_a)
        start_idx(1, idx_b, si_b)

        @pl.loop(0, chunks, step=2)
        def _pooled(c):
            wait_idx(idx_b, si_b)
            fire_gathers(c + 1, idx_b, rows_b, sg_b)

            wait_gathers(rows_a, sg_a)

            @pl.when(c + 2 < chunks)
            def _():
                start_idx(c + 2, idx_a, si_a)

            @pl.when(c >= 2)
            def _():
                wait_out(out_a, so_a)
            process(c, rows_a, out_a, so_a)

            @pl.when(c + 2 < chunks)
            def _():
                wait_idx(idx_a, si_a)
                fire_gathers(c + 2, idx_a, rows_a, sg_a)

            wait_gathers(rows_b, sg_b)

            @pl.when(c + 3 < chunks)
            def _():
                start_idx(c + 3, idx_b, si_b)

            @pl.when(c >= 2)
            def _():
                wait_out(out_b, so_b)
            process(c + 1, rows_b, out_b, so_b)

        wait_out(out_a, so_a)
        wait_out(out_b, so_b)

    section(lt_hbm, lidx_hbm, 0, _NLT)
    section(gt_hbm, gidx_hbm, _NLT, _NGT)

    @pl.loop(0, _SHOW_CHUNKS)
    def _show(j):
        rrow = wid * _SHOW_CHUNKS + j
        s0 = pl.multiple_of(rrow * 128, 128)
        pltpu.sync_copy(sidx_hbm.at[pl.ds(s0, 128)], sidx_v)
        pltpu.async_copy(show_hbm.at[sidx_v],
                         rows_a.at[pl.ds(0, 128)], sem).wait()
        pltpu.sync_copy(rows_a.at[pl.ds(0, 128)],
                        out_hbm.at[_NPOOL, pl.ds(s0, 128)])


def _tc_body(e_ref, wh_ref, wl_ref, b_ref, o_ref, zh_ref, zl_ref):
    es = [e_ref[i] for i in range(_NE)]

    def put(col, val):
        hi = val.astype(jnp.bfloat16)
        lo = (val - hi.astype(jnp.float32)).astype(jnp.bfloat16)
        zh_ref[:, pl.ds(col, _D)] = hi
        zl_ref[:, pl.ds(col, _D)] = lo

    for p, (i, j) in enumerate(_PAIRS):
        put(p * _D, es[i] * es[j])
    for t in range(_NE):
        put(_NPAIR * _D + t * _D, es[t])
    zh, zl = zh_ref[...], zl_ref[...]
    o_ref[...] = (
        jnp.dot(zh, wh_ref[...], preferred_element_type=jnp.float32)
        + jnp.dot(zh, wl_ref[...], preferred_element_type=jnp.float32)
        + jnp.dot(zl, wh_ref[...], preferred_element_type=jnp.float32)
        + b_ref[...]
    )


def _sc_embed(lt_tables, gt_tables, show_table, lidx, gidx, sidx):
    mesh = plsc.VectorSubcoreMesh(core_axis_name="c", subcore_axis_name="s",
                                  num_cores=_NC, num_subcores=_NS)
    run = pl.kernel(
        _sc_body,
        out_type=jax.ShapeDtypeStruct((_NE, _B, _D), jnp.float32),
        mesh=mesh,
        compiler_params=pltpu.CompilerParams(use_tc_tiling_on_sc=False),
        scratch_types=[
            pltpu.VMEM((_CHUNK_BAGS * _L,), jnp.int32),
            pltpu.VMEM((_CHUNK_BAGS * _L,), jnp.int32),
            pltpu.VMEM((128,), jnp.int32),
            pltpu.VMEM((_CHUNK_BAGS * _L, _D), jnp.float32),
            pltpu.VMEM((_CHUNK_BAGS * _L, _D), jnp.float32),
            pltpu.VMEM((_CHUNK_BAGS, _D), jnp.float32),
            pltpu.VMEM((_CHUNK_BAGS, _D), jnp.float32),
            pltpu.SemaphoreType.DMA,
            pltpu.SemaphoreType.DMA,
            pltpu.SemaphoreType.DMA,
            pltpu.SemaphoreType.DMA,
            pltpu.SemaphoreType.DMA,
            pltpu.SemaphoreType.DMA,
            pltpu.SemaphoreType.DMA,
        ],
    )
    return run(lt_tables, gt_tables, show_table, lidx, gidx, sidx)


def _tc_interact(embeds, w2, b2, bt=512):
    wh = w2.astype(jnp.bfloat16)
    wl = (w2 - wh.astype(jnp.float32)).astype(jnp.bfloat16)
    return pl.pallas_call(
        _tc_body,
        grid=(_B // bt,),
        in_specs=[
            pl.BlockSpec((_NE, bt, _D), lambda i: (0, i, 0)),
            pl.BlockSpec((_K2, _NLBL), lambda i: (0, 0)),
            pl.BlockSpec((_K2, _NLBL), lambda i: (0, 0)),
            pl.BlockSpec((1, _NLBL), lambda i: (0, 0)),
        ],
        out_specs=pl.BlockSpec((bt, _NLBL), lambda i: (i, 0)),
        out_shape=jax.ShapeDtypeStruct((_B, _NLBL), jnp.float32),
        scratch_shapes=[pltpu.VMEM((bt, _K2), jnp.bfloat16),
                        pltpu.VMEM((bt, _K2), jnp.bfloat16)],
    )(embeds, wh, wl, b2)


def kernel(lt_inputs, lt_offsets, gt_inputs, gt_offsets, show_ids,
           lt_tables, gt_tables, show_table, lin_w, lin_b):
    embeds = _sc_embed(lt_tables, gt_tables, show_table,
                       lt_inputs.reshape(-1), gt_inputs.reshape(-1),
                       show_ids)

    w2 = jnp.concatenate(
        [jnp.repeat(lin_w[:, :_NPAIR], _D, axis=1) * (1.0 / _D),
         lin_w[:, _NPAIR:]], axis=1).T
    out2 = _tc_interact(embeds, w2, lin_b.reshape(1, _NLBL))
    return tuple(out2[:, k] for k in range(_NLBL))


# speedup vs baseline: 24.4780x; 1.1864x over previous
import functools

import jax
import jax.numpy as jnp
from jax import lax
from jax.experimental import pallas as pl
from jax.experimental.pallas import tpu as pltpu
from jax.experimental.pallas import tpu_sc as plsc

_HASH = 21000
_NLT = 6
_NGT = 5
_NPOOL = _NLT + _NGT
_NE = _NPOOL + 1
_NLBL = 6
_B = 16384
_L = 20
_D = 64
_NPAIR = _NE * (_NE + 1) // 2
_K2 = _NPAIR * _D + _NE * _D

_NC, _NS = 2, 16
_NW = _NC * _NS
_NSPLIT = 2
_BH = _B // _NSPLIT
_BH_SHIFT = 13
_CHUNK_BAGS = 32
_IDX_ROWS = _CHUNK_BAGS * _L // 128
_SHOW_CHUNKS = _BH // _NW // 128

_PAIRS = [(i, j) for i in range(_NE) for j in range(i, _NE)]


def _sc_body(off, lt_hbm, gt_hbm, show_hbm, lidx_hbm, gidx_hbm, sidx_hbm, out_hbm,
             idx_a, idx_b, sidx_v, rows_a, rows_b, out_a, out_b,
             si_a, si_b, sg_a, sg_b, so_a, so_b, sem):
    wid = lax.axis_index("s") * _NC + lax.axis_index("c")
    nidx = _CHUNK_BAGS * _L

    def wait_idx(idx_v, si):
        pltpu.make_async_copy(lidx_hbm.at[pl.ds(0, nidx)], idx_v, si).wait()

    def wait_gathers(rows_v, sg):
        pltpu.make_async_copy(show_hbm.at[pl.ds(0, nidx)], rows_v, sg).wait()

    def wait_out(out_v, so):
        pltpu.make_async_copy(out_v, out_hbm.at[0, pl.ds(0, _CHUNK_BAGS)], so).wait()

    def section(tab_hbm, idx_hbm, plane_base, ntab):
        bags_per_w = ntab * _BH // _NW
        chunks = bags_per_w // _CHUNK_BAGS

        def start_idx(c, idx_v, si):
            g0 = wid * bags_per_w + c * _CHUNK_BAGS
            t = g0 >> _BH_SHIFT
            r = g0 & (_BH - 1)
            i0 = pl.multiple_of((t * _B + off + r) * _L, nidx)
            pltpu.async_copy(idx_hbm.at[pl.ds(i0, nidx)], idx_v, si)

        def fire_gathers(c, idx_v, rows_v, sg):
            g0 = wid * bags_per_w + c * _CHUNK_BAGS
            t = g0 >> _BH_SHIFT
            for j in range(_IDX_ROWS):
                pltpu.async_copy(tab_hbm.at[t].at[idx_v.at[pl.ds(j * 128, 128)]],
                                 rows_v.at[pl.ds(j * 128, 128)], sg)

        def process(c, rows_v, out_v, so):
            @pl.loop(0, _CHUNK_BAGS)
            def _bag(bi):
                rr = bi * _L
                accs = [rows_v[rr, pl.ds(cc * 16, 16)] for cc in range(4)]
                for l in range(1, _L):
                    accs = [jnp.maximum(accs[cc], rows_v[rr + l, pl.ds(cc * 16, 16)])
                            for cc in range(4)]
                for cc in range(4):
                    out_v[bi, pl.ds(cc * 16, 16)] = accs[cc]

            g0 = wid * bags_per_w + c * _CHUNK_BAGS
            t = g0 >> _BH_SHIFT
            b0 = pl.multiple_of(g0 & (_BH - 1), _CHUNK_BAGS)
            pltpu.async_copy(out_v, out_hbm.at[plane_base + t, pl.ds(b0, _CHUNK_BAGS)], so)

        start_idx(0, idx_a, si_a)
        wait_idx(idx_a, si_a)
        fire_gathers(0, idx_a, rows_a, sg_a)
        start_idx(1, idx_b, si_b)

        @pl.loop(0, chunks, step=2)
        def _pooled(c):
            wait_idx(idx_b, si_b)
            fire_gathers(c + 1, idx_b, rows_b, sg_b)

            wait_gathers(rows_a, sg_a)

            @pl.when(c + 2 < chunks)
            def _():
                start_idx(c + 2, idx_a, si_a)

            @pl.when(c >= 2)
            def _():
                wait_out(out_a, so_a)
            process(c, rows_a, out_a, so_a)

            @pl.when(c + 2 < chunks)
            def _():
                wait_idx(idx_a, si_a)
                fire_gathers(c + 2, idx_a, rows_a, sg_a)

            wait_gathers(rows_b, sg_b)

            @pl.when(c + 3 < chunks)
            def _():
                start_idx(c + 3, idx_b, si_b)

            @pl.when(c >= 2)
            def _():
                wait_out(out_b, so_b)
            process(c + 1, rows_b, out_b, so_b)

        wait_out(out_a, so_a)
        wait_out(out_b, so_b)

    section(lt_hbm, lidx_hbm, 0, _NLT)
    section(gt_hbm, gidx_hbm, _NLT, _NGT)

    @pl.loop(0, _SHOW_CHUNKS)
    def _show(j):
        rrow = wid * _SHOW_CHUNKS + j
        s0 = pl.multiple_of(rrow * 128, 128)
        src0 = pl.multiple_of(off + rrow * 128, 128)
        pltpu.sync_copy(sidx_hbm.at[pl.ds(src0, 128)], sidx_v)
        pltpu.async_copy(show_hbm.at[sidx_v],
                         rows_a.at[pl.ds(0, 128)], sem).wait()
        pltpu.sync_copy(rows_a.at[pl.ds(0, 128)],
                        out_hbm.at[_NPOOL, pl.ds(s0, 128)])


def _tc_body(e_ref, wh_ref, wl_ref, b_ref, o_ref, zh_ref, zl_ref):
    es = [e_ref[i] for i in range(_NE)]

    def put(col, val):
        hi = val.astype(jnp.bfloat16)
        lo = (val - hi.astype(jnp.float32)).astype(jnp.bfloat16)
        zh_ref[:, pl.ds(col, _D)] = hi
        zl_ref[:, pl.ds(col, _D)] = lo

    for p, (i, j) in enumerate(_PAIRS):
        put(p * _D, es[i] * es[j])
    for t in range(_NE):
        put(_NPAIR * _D + t * _D, es[t])
    zh, zl = zh_ref[...], zl_ref[...]
    o_ref[...] = (
        jnp.dot(zh, wh_ref[...], preferred_element_type=jnp.float32)
        + jnp.dot(zh, wl_ref[...], preferred_element_type=jnp.float32)
        + jnp.dot(zl, wh_ref[...], preferred_element_type=jnp.float32)
        + b_ref[...]
    )


def _sc_embed(off, lt_tables, gt_tables, show_table, lidx, gidx, sidx):
    mesh = plsc.VectorSubcoreMesh(core_axis_name="c", subcore_axis_name="s",
                                  num_cores=_NC, num_subcores=_NS)
    run = pl.kernel(
        functools.partial(_sc_body, off),
        out_type=jax.ShapeDtypeStruct((_NE, _BH, _D), jnp.float32),
        mesh=mesh,
        compiler_params=pltpu.CompilerParams(use_tc_tiling_on_sc=False),
        scratch_types=[
            pltpu.VMEM((_CHUNK_BAGS * _L,), jnp.int32),
            pltpu.VMEM((_CHUNK_BAGS * _L,), jnp.int32),
            pltpu.VMEM((128,), jnp.int32),
            pltpu.VMEM((_CHUNK_BAGS * _L, _D), jnp.float32),
            pltpu.VMEM((_CHUNK_BAGS * _L, _D), jnp.float32),
            pltpu.VMEM((_CHUNK_BAGS, _D), jnp.float32),
            pltpu.VMEM((_CHUNK_BAGS, _D), jnp.float32),
            pltpu.SemaphoreType.DMA,
            pltpu.SemaphoreType.DMA,
            pltpu.SemaphoreType.DMA,
            pltpu.SemaphoreType.DMA,
            pltpu.SemaphoreType.DMA,
            pltpu.SemaphoreType.DMA,
            pltpu.SemaphoreType.DMA,
        ],
    )
    return run(lt_tables, gt_tables, show_table, lidx, gidx, sidx)


def _tc_interact(embeds, wh, wl, b2, bt=512):
    return pl.pallas_call(
        _tc_body,
        grid=(_BH // bt,),
        in_specs=[
            pl.BlockSpec((_NE, bt, _D), lambda i: (0, i, 0)),
            pl.BlockSpec((_K2, _NLBL), lambda i: (0, 0)),
            pl.BlockSpec((_K2, _NLBL), lambda i: (0, 0)),
            pl.BlockSpec((1, _NLBL), lambda i: (0, 0)),
        ],
        out_specs=pl.BlockSpec((bt, _NLBL), lambda i: (i, 0)),
        out_shape=jax.ShapeDtypeStruct((_BH, _NLBL), jnp.float32),
        scratch_shapes=[pltpu.VMEM((bt, _K2), jnp.bfloat16),
                        pltpu.VMEM((bt, _K2), jnp.bfloat16)],
    )(embeds, wh, wl, b2)


def kernel(lt_inputs, lt_offsets, gt_inputs, gt_offsets, show_ids,
           lt_tables, gt_tables, show_table, lin_w, lin_b):
    lidx = lt_inputs.reshape(-1)
    gidx = gt_inputs.reshape(-1)
    w2 = jnp.concatenate(
        [jnp.repeat(lin_w[:, :_NPAIR], _D, axis=1) * (1.0 / _D),
         lin_w[:, _NPAIR:]], axis=1).T
    wh = w2.astype(jnp.bfloat16)
    wl = (w2 - wh.astype(jnp.float32)).astype(jnp.bfloat16)
    b2 = lin_b.reshape(1, _NLBL)

    outs = []
    for s in range(_NSPLIT):
        embeds = _sc_embed(s * _BH, lt_tables, gt_tables, show_table,
                           lidx, gidx, show_ids)
        outs.append(_tc_interact(embeds, wh, wl, b2))
    out2 = jnp.concatenate(outs, axis=0)
    return tuple(out2[:, k] for k in range(_NLBL))


# speedup vs baseline: 25.9384x; 1.0597x over previous
import functools

import jax
import jax.numpy as jnp
from jax import lax
from jax.experimental import pallas as pl
from jax.experimental.pallas import tpu as pltpu
from jax.experimental.pallas import tpu_sc as plsc

_HASH = 21000
_NLT = 6
_NGT = 5
_NPOOL = _NLT + _NGT
_NE = _NPOOL + 1
_NLBL = 6
_B = 16384
_L = 20
_D = 64
_NPAIR = _NE * (_NE + 1) // 2
_K2 = _NPAIR * _D + _NE * _D

_NC, _NS = 2, 16
_NW = _NC * _NS
_NSPLIT = 4
_BH = _B // _NSPLIT
_BH_SHIFT = 12
_CHUNK_BAGS = 32
_IDX_ROWS = _CHUNK_BAGS * _L // 128
_SHOW_CHUNKS = _BH // _NW // 128

_PAIRS = [(i, j) for i in range(_NE) for j in range(i, _NE)]


def _sc_body(off, lt_hbm, gt_hbm, show_hbm, lidx_hbm, gidx_hbm, sidx_hbm, out_hbm,
             idx_a, idx_b, sidx_v, rows_a, rows_b, out_a, out_b,
             si_a, si_b, sg_a, sg_b, so_a, so_b, sem):
    wid = lax.axis_index("s") * _NC + lax.axis_index("c")
    nidx = _CHUNK_BAGS * _L

    def wait_idx(idx_v, si):
        pltpu.make_async_copy(lidx_hbm.at[pl.ds(0, nidx)], idx_v, si).wait()

    def wait_gathers(rows_v, sg):
        pltpu.make_async_copy(show_hbm.at[pl.ds(0, nidx)], rows_v, sg).wait()

    def wait_out(out_v, so):
        pltpu.make_async_copy(out_v, out_hbm.at[0, pl.ds(0, _CHUNK_BAGS)], so).wait()

    def section(tab_hbm, idx_hbm, plane_base, ntab):
        bags_per_w = ntab * _BH // _NW
        chunks = bags_per_w // _CHUNK_BAGS

        def start_idx(c, idx_v, si):
            g0 = wid * bags_per_w + c * _CHUNK_BAGS
            t = g0 >> _BH_SHIFT
            r = g0 & (_BH - 1)
            i0 = pl.multiple_of((t * _B + off + r) * _L, nidx)
            pltpu.async_copy(idx_hbm.at[pl.ds(i0, nidx)], idx_v, si)

        def fire_gathers(c, idx_v, rows_v, sg):
            g0 = wid * bags_per_w + c * _CHUNK_BAGS
            t = g0 >> _BH_SHIFT
            for j in range(_IDX_ROWS):
                pltpu.async_copy(tab_hbm.at[t].at[idx_v.at[pl.ds(j * 128, 128)]],
                                 rows_v.at[pl.ds(j * 128, 128)], sg)

        def process(c, rows_v, out_v, so):
            @pl.loop(0, _CHUNK_BAGS)
            def _bag(bi):
                rr = bi * _L
                accs = [rows_v[rr, pl.ds(cc * 16, 16)] for cc in range(4)]
                for l in range(1, _L):
                    accs = [jnp.maximum(accs[cc], rows_v[rr + l, pl.ds(cc * 16, 16)])
                            for cc in range(4)]
                for cc in range(4):
                    out_v[bi, pl.ds(cc * 16, 16)] = accs[cc]

            g0 = wid * bags_per_w + c * _CHUNK_BAGS
            t = g0 >> _BH_SHIFT
            b0 = pl.multiple_of(g0 & (_BH - 1), _CHUNK_BAGS)
            pltpu.async_copy(out_v, out_hbm.at[plane_base + t, pl.ds(b0, _CHUNK_BAGS)], so)

        start_idx(0, idx_a, si_a)
        wait_idx(idx_a, si_a)
        fire_gathers(0, idx_a, rows_a, sg_a)
        start_idx(1, idx_b, si_b)

        @pl.loop(0, chunks, step=2)
        def _pooled(c):
            wait_idx(idx_b, si_b)
            fire_gathers(c + 1, idx_b, rows_b, sg_b)

            wait_gathers(rows_a, sg_a)

            @pl.when(c + 2 < chunks)
            def _():
                start_idx(c + 2, idx_a, si_a)

            @pl.when(c >= 2)
            def _():
                wait_out(out_a, so_a)
            process(c, rows_a, out_a, so_a)

            @pl.when(c + 2 < chunks)
            def _():
                wait_idx(idx_a, si_a)
                fire_gathers(c + 2, idx_a, rows_a, sg_a)

            wait_gathers(rows_b, sg_b)

            @pl.when(c + 3 < chunks)
            def _():
                start_idx(c + 3, idx_b, si_b)

            @pl.when(c >= 2)
            def _():
                wait_out(out_b, so_b)
            process(c + 1, rows_b, out_b, so_b)

        wait_out(out_a, so_a)
        wait_out(out_b, so_b)

    section(lt_hbm, lidx_hbm, 0, _NLT)
    section(gt_hbm, gidx_hbm, _NLT, _NGT)

    @pl.loop(0, _SHOW_CHUNKS)
    def _show(j):
        rrow = wid * _SHOW_CHUNKS + j
        s0 = pl.multiple_of(rrow * 128, 128)
        src0 = pl.multiple_of(off + rrow * 128, 128)
        pltpu.sync_copy(sidx_hbm.at[pl.ds(src0, 128)], sidx_v)
        pltpu.async_copy(show_hbm.at[sidx_v],
                         rows_a.at[pl.ds(0, 128)], sem).wait()
        pltpu.sync_copy(rows_a.at[pl.ds(0, 128)],
                        out_hbm.at[_NPOOL, pl.ds(s0, 128)])


def _tc_body(e_ref, wh_ref, wl_ref, b_ref, o_ref, zh_ref, zl_ref):
    es = [e_ref[i] for i in range(_NE)]

    def put(col, val):
        hi = val.astype(jnp.bfloat16)
        lo = (val - hi.astype(jnp.float32)).astype(jnp.bfloat16)
        zh_ref[:, pl.ds(col, _D)] = hi
        zl_ref[:, pl.ds(col, _D)] = lo

    for p, (i, j) in enumerate(_PAIRS):
        put(p * _D, es[i] * es[j])
    for t in range(_NE):
        put(_NPAIR * _D + t * _D, es[t])
    zh, zl = zh_ref[...], zl_ref[...]
    o_ref[...] = (
        jnp.dot(zh, wh_ref[...], preferred_element_type=jnp.float32)
        + jnp.dot(zh, wl_ref[...], preferred_element_type=jnp.float32)
        + jnp.dot(zl, wh_ref[...], preferred_element_type=jnp.float32)
        + b_ref[...]
    )


def _sc_embed(off, lt_tables, gt_tables, show_table, lidx, gidx, sidx):
    mesh = plsc.VectorSubcoreMesh(core_axis_name="c", subcore_axis_name="s",
                                  num_cores=_NC, num_subcores=_NS)
    run = pl.kernel(
        functools.partial(_sc_body, off),
        out_type=jax.ShapeDtypeStruct((_NE, _BH, _D), jnp.float32),
        mesh=mesh,
        compiler_params=pltpu.CompilerParams(use_tc_tiling_on_sc=False),
        scratch_types=[
            pltpu.VMEM((_CHUNK_BAGS * _L,), jnp.int32),
            pltpu.VMEM((_CHUNK_BAGS * _L,), jnp.int32),
            pltpu.VMEM((128,), jnp.int32),
            pltpu.VMEM((_CHUNK_BAGS * _L, _D), jnp.float32),
            pltpu.VMEM((_CHUNK_BAGS * _L, _D), jnp.float32),
            pltpu.VMEM((_CHUNK_BAGS, _D), jnp.float32),
            pltpu.VMEM((_CHUNK_BAGS, _D), jnp.float32),
            pltpu.SemaphoreType.DMA,
            pltpu.SemaphoreType.DMA,
            pltpu.SemaphoreType.DMA,
            pltpu.SemaphoreType.DMA,
            pltpu.SemaphoreType.DMA,
            pltpu.SemaphoreType.DMA,
            pltpu.SemaphoreType.DMA,
        ],
    )
    return run(lt_tables, gt_tables, show_table, lidx, gidx, sidx)


def _tc_interact(embeds, wh, wl, b2, bt=512):
    return pl.pallas_call(
        _tc_body,
        grid=(_BH // bt,),
        in_specs=[
            pl.BlockSpec((_NE, bt, _D), lambda i: (0, i, 0)),
            pl.BlockSpec((_K2, _NLBL), lambda i: (0, 0)),
            pl.BlockSpec((_K2, _NLBL), lambda i: (0, 0)),
            pl.BlockSpec((1, _NLBL), lambda i: (0, 0)),
        ],
        out_specs=pl.BlockSpec((bt, _NLBL), lambda i: (i, 0)),
        out_shape=jax.ShapeDtypeStruct((_BH, _NLBL), jnp.float32),
        scratch_shapes=[pltpu.VMEM((bt, _K2), jnp.bfloat16),
                        pltpu.VMEM((bt, _K2), jnp.bfloat16)],
    )(embeds, wh, wl, b2)


def kernel(lt_inputs, lt_offsets, gt_inputs, gt_offsets, show_ids,
           lt_tables, gt_tables, show_table, lin_w, lin_b):
    lidx = lt_inputs.reshape(-1)
    gidx = gt_inputs.reshape(-1)
    w2 = jnp.concatenate(
        [jnp.repeat(lin_w[:, :_NPAIR], _D, axis=1) * (1.0 / _D),
         lin_w[:, _NPAIR:]], axis=1).T
    wh = w2.astype(jnp.bfloat16)
    wl = (w2 - wh.astype(jnp.float32)).astype(jnp.bfloat16)
    b2 = lin_b.reshape(1, _NLBL)

    outs = []
    for s in range(_NSPLIT):
        embeds = _sc_embed(s * _BH, lt_tables, gt_tables, show_table,
                           lidx, gidx, show_ids)
        outs.append(_tc_interact(embeds, wh, wl, b2))
    out2 = jnp.concatenate(outs, axis=0)
    return tuple(out2[:, k] for k in range(_NLBL))
